# baseline (speedup 1.0000x reference)
import functools
import math

import jax
import jax.numpy as jnp
from jax import lax
from jax.experimental import pallas as pl
from jax.experimental.pallas import tpu as pltpu
from jax.experimental.pallas import tpu_sc as plsc

D_MODEL = 64
SCALE = math.sqrt(D_MODEL)

NUM_CORES = 2
NUM_SUBCORES = 16
NUM_WORKERS = NUM_CORES * NUM_SUBCORES
LANES = 16

IDX_PER_STREAM = 128
STREAMS_PER_CHUNK = 4
CHUNK = IDX_PER_STREAM * STREAMS_PER_CHUNK
ROWS_PER_ITER = 4


@functools.partial(jax.jit, static_argnums=(2, 3))
def _emb_lookup(x1d, table, n_idx, d_model):
    per_worker = n_idx // NUM_WORKERS
    n_chunks = per_worker // CHUNK
    mesh = plsc.VectorSubcoreMesh(core_axis_name="c", subcore_axis_name="s")

    @functools.partial(
        pl.kernel,
        out_type=jax.ShapeDtypeStruct((n_idx, d_model), jnp.float32),
        mesh=mesh,
        scratch_types=[
            pltpu.VMEM((CHUNK,), jnp.int32),
            pltpu.VMEM((CHUNK, d_model), jnp.float32),
            pltpu.SemaphoreType.DMA,
        ],
        compiler_params=pltpu.CompilerParams(use_tc_tiling_on_sc=False),
    )
    def emb_kernel(x_hbm, tab_hbm, out_hbm, idx_v, rows_v, sem):
        wid = lax.axis_index("s") * NUM_CORES + lax.axis_index("c")
        base = wid * per_worker

        def chunk_body(c, carry):
            off = base + c * CHUNK
            pltpu.sync_copy(x_hbm.at[pl.ds(off, CHUNK)], idx_v)
            copies = [
                pltpu.async_copy(
                    tab_hbm.at[idx_v.at[pl.ds(j * IDX_PER_STREAM,
                                              IDX_PER_STREAM)]],
                    rows_v.at[pl.ds(j * IDX_PER_STREAM, IDX_PER_STREAM)],
                    sem,
                )
                for j in range(STREAMS_PER_CHUNK)
            ]
            for cp in copies:
                cp.wait()

            def scale_body(i, carry2):
                r0 = i * ROWS_PER_ITER
                for dr in range(ROWS_PER_ITER):
                    for j in range(d_model // LANES):
                        sl = pl.ds(j * LANES, LANES)
                        rows_v[r0 + dr, sl] = rows_v[r0 + dr, sl] * SCALE
                return carry2

            lax.fori_loop(0, CHUNK // ROWS_PER_ITER, scale_body, 0,
                          unroll=False)

            pltpu.sync_copy(rows_v, out_hbm.at[pl.ds(off, CHUNK)])
            return carry

        lax.fori_loop(0, n_chunks, chunk_body, 0, unroll=False)

    return emb_kernel(x1d, table)


def kernel(x, table):
    b, s = x.shape
    n_idx = b * s
    x1d = x.reshape(n_idx)
    out = _emb_lookup(x1d, table, n_idx, table.shape[1])
    return out.reshape(b, s, table.shape[1])


# speedup vs baseline: 1.0939x; 1.0939x over previous
import functools
import math

import jax
import jax.numpy as jnp
from jax import lax
from jax.experimental import pallas as pl
from jax.experimental.pallas import tpu as pltpu
from jax.experimental.pallas import tpu_sc as plsc

D_MODEL = 64
SCALE = math.sqrt(D_MODEL)

NUM_CORES = 2
NUM_SUBCORES = 16
NUM_WORKERS = NUM_CORES * NUM_SUBCORES
LANES = 16

IDX_PER_STREAM = 128
STREAMS_PER_CHUNK = 4
CHUNK = IDX_PER_STREAM * STREAMS_PER_CHUNK
ROWS_PER_ITER = 8


@functools.partial(jax.jit, static_argnums=(2, 3))
def _emb_lookup(x1d, table, n_idx, d_model):
    per_worker = n_idx // NUM_WORKERS
    n_chunks = per_worker // CHUNK
    assert n_chunks % 2 == 0 and n_chunks >= 4
    n_pairs = n_chunks // 2
    mesh = plsc.VectorSubcoreMesh(core_axis_name="c", subcore_axis_name="s")

    @functools.partial(
        pl.kernel,
        out_type=jax.ShapeDtypeStruct((n_idx, d_model), jnp.float32),
        mesh=mesh,
        scratch_types=[
            pltpu.VMEM((per_worker,), jnp.int32),
            pltpu.VMEM((2, CHUNK, d_model), jnp.float32),
            pltpu.SemaphoreType.DMA,
            pltpu.SemaphoreType.DMA,
            pltpu.SemaphoreType.DMA,
            pltpu.SemaphoreType.DMA,
        ],
        compiler_params=pltpu.CompilerParams(use_tc_tiling_on_sc=False),
    )
    def emb_kernel(x_hbm, tab_hbm, out_hbm, idx_all, rows_v,
                   gsem0, gsem1, wsem0, wsem1):
        wid = lax.axis_index("s") * NUM_CORES + lax.axis_index("c")
        base = wid * per_worker
        gsems = (gsem0, gsem1)
        wsems = (wsem0, wsem1)

        def fire_gathers(c, buf):
            for j in range(STREAMS_PER_CHUNK):
                pltpu.async_copy(
                    tab_hbm.at[idx_all.at[pl.ds(
                        c * CHUNK + j * IDX_PER_STREAM, IDX_PER_STREAM)]],
                    rows_v.at[buf, pl.ds(j * IDX_PER_STREAM, IDX_PER_STREAM)],
                    gsems[buf],
                )

        def drain_gathers(c, buf):
            pltpu.make_async_copy(
                out_hbm.at[pl.ds(base + c * CHUNK, CHUNK)],
                rows_v.at[buf],
                gsems[buf],
            ).wait()

        def scale_buf(buf):
            def scale_body(i, carry):
                r0 = i * ROWS_PER_ITER
                for dr in range(ROWS_PER_ITER):
                    for j in range(d_model // LANES):
                        sl = pl.ds(j * LANES, LANES)
                        rows_v[buf, r0 + dr, sl] = (
                            rows_v[buf, r0 + dr, sl] * SCALE)
                return carry
            lax.fori_loop(0, CHUNK // ROWS_PER_ITER, scale_body, 0,
                          unroll=False)

        def wb_desc(c, buf):
            return pltpu.make_async_copy(
                rows_v.at[buf],
                out_hbm.at[pl.ds(base + c * CHUNK, CHUNK)],
                wsems[buf],
            )

        pltpu.sync_copy(x_hbm.at[pl.ds(base, per_worker)], idx_all)
        fire_gathers(0, 0)

        def pair_body(p, carry):
            c0 = 2 * p
            @pl.when(p >= 1)
            def _():
                wb_desc(c0 - 1, 1).wait()
            fire_gathers(c0 + 1, 1)
            drain_gathers(c0, 0)
            scale_buf(0)
            wb_desc(c0, 0).start()

            c1 = c0 + 1
            @pl.when(p < n_pairs - 1)
            def _():
                wb_desc(c0, 0).wait()
                fire_gathers(c1 + 1, 0)
            drain_gathers(c1, 1)
            scale_buf(1)
            wb_desc(c1, 1).start()
            return carry

        lax.fori_loop(0, n_pairs, pair_body, 0, unroll=False)
        wb_desc(n_chunks - 2, 0).wait()
        wb_desc(n_chunks - 1, 1).wait()

    return emb_kernel(x1d, table)


def kernel(x, table):
    b, s = x.shape
    n_idx = b * s
    x1d = x.reshape(n_idx)
    out = _emb_lookup(x1d, table, n_idx, table.shape[1])
    return out.reshape(b, s, table.shape[1])
